# baseline (speedup 1.0000x reference)
import functools

import jax
import jax.numpy as jnp
from jax import lax
from jax.experimental import pallas as pl
from jax.experimental.pallas import tpu as pltpu
from jax.experimental.pallas import tpu_sc as plsc

NC = 1
NS = 16
NW = NC * NS
BATCH = 16384
LATENT = 64
CHUNK = BATCH // NW
NIDX = 8
IDXW = CHUNK // NIDX
HALF = NIDX // 2

_mesh = plsc.VectorSubcoreMesh(core_axis_name="c", subcore_axis_name="s",
                               num_cores=NC)

_scratch_types = [
    pltpu.VMEM((NIDX, IDXW), jnp.int32),
    pltpu.VMEM((NIDX, IDXW), jnp.int32),
    pltpu.VMEM((HALF, IDXW, LATENT), jnp.float32),
    pltpu.VMEM((HALF, IDXW, LATENT), jnp.float32),
    pltpu.VMEM((NIDX, IDXW), jnp.float32),
    pltpu.VMEM((NIDX, IDXW), jnp.float32),
    pltpu.VMEM((CHUNK,), jnp.float32),
    pltpu.SemaphoreType.DMA,
]


def _mf_body(users_hbm, items_hbm, p_hbm, q_hbm, bu_hbm, bi_hbm, out_hbm,
             uidx, iidx, pm, qm, bu, bi, outb, sem):
    wid = lax.axis_index("s")

    pltpu.sync_copy(users_hbm.at[wid], uidx)
    pltpu.sync_copy(items_hbm.at[wid], iidx)

    bias_copies = []
    for j in range(NIDX):
        bias_copies.append(pltpu.async_copy(bu_hbm.at[uidx.at[j]], bu.at[j], sem))
        bias_copies.append(pltpu.async_copy(bi_hbm.at[iidx.at[j]], bi.at[j], sem))

    lane = lax.iota(jnp.int32, 16)
    perms = [lane ^ s for s in (1, 2, 4, 8)]
    picks = [(lane & s) == 0 for s in (1, 2, 4, 8)]

    def _take(v, idx):
        return jnp.take_along_axis(v, idx, axis=0)

    for c in bias_copies:
        c.wait()

    for h in range(2):
        copies = []
        for j in range(HALF):
            copies.append(pltpu.async_copy(
                p_hbm.at[uidx.at[h * HALF + j]], pm.at[j], sem))
            copies.append(pltpu.async_copy(
                q_hbm.at[iidx.at[h * HALF + j]], qm.at[j], sem))
        for c in copies:
            c.wait()

        def blk_body(blk, carry):
            j = blk // (IDXW // 16)
            off = (blk % (IDXW // 16)) * 16
            vs = []
            for r in range(16):
                row = off + r
                acc = pm[j, row, pl.ds(0, 16)] * qm[j, row, pl.ds(0, 16)]
                for k in range(1, LATENT // 16):
                    acc = acc + pm[j, row, pl.ds(16 * k, 16)] * qm[j, row, pl.ds(16 * k, 16)]
                vs.append(acc)
            for lvl in range(4):
                nxt = []
                for i in range(0, len(vs), 2):
                    a, b = vs[i], vs[i + 1]
                    ap = a + _take(a, perms[lvl])
                    bp = b + _take(b, perms[lvl])
                    nxt.append(jnp.where(picks[lvl], ap, bp))
                vs = nxt
            jj = h * HALF + j
            out_v = (vs[0] + bu[jj, pl.ds(off, 16)] + bi[jj, pl.ds(off, 16)])
            outb[pl.ds(h * (CHUNK // 2) + blk * 16, 16)] = out_v
            return carry

        lax.fori_loop(0, HALF * IDXW // 16, blk_body, 0)

    pltpu.sync_copy(outb, out_hbm.at[pl.ds(wid * CHUNK, CHUNK)])


_mf_kernel = functools.partial(
    pl.kernel,
    out_type=jax.ShapeDtypeStruct((BATCH,), jnp.float32),
    mesh=_mesh,
    scratch_types=_scratch_types,
    compiler_params=pltpu.CompilerParams(use_tc_tiling_on_sc=False),
)(_mf_body)


def kernel(users, items, P, Q, user_bias, item_bias):
    users_r = users.reshape(NW, NIDX, IDXW)
    items_r = items.reshape(NW, NIDX, IDXW)
    bu_flat = user_bias.reshape(-1)
    bi_flat = item_bias.reshape(-1)
    return _mf_kernel(users_r, items_r, P, Q, bu_flat, bi_flat)


# speedup vs baseline: 1.4251x; 1.4251x over previous
import functools

import jax
import jax.numpy as jnp
from jax import lax
from jax.experimental import pallas as pl
from jax.experimental.pallas import tpu as pltpu
from jax.experimental.pallas import tpu_sc as plsc

NC = 2
NS = 16
NW = NC * NS
BATCH = 16384
LATENT = 64
CHUNK = BATCH // NW
NIDX = 4
IDXW = CHUNK // NIDX
GRP = 32
NGRP = CHUNK // GRP

_mesh = plsc.VectorSubcoreMesh(core_axis_name="c", subcore_axis_name="s")

_dot_scratch = [
    pltpu.VMEM((NIDX, IDXW), jnp.int32),
    pltpu.VMEM((NIDX, IDXW), jnp.int32),
    pltpu.VMEM((GRP, 8, LATENT), jnp.float32),
    pltpu.VMEM((GRP, 8, LATENT), jnp.float32),
    pltpu.VMEM((CHUNK,), jnp.float32),
    pltpu.SemaphoreType.DMA,
]


def _dot_body(uid_hbm, iid_hbm, p_hbm, q_hbm, out_hbm,
              uid, iid, pg, qg, outb, sem):
    wid = lax.axis_index("s") * NC + lax.axis_index("c")

    pltpu.sync_copy(uid_hbm.at[wid], uid)
    pltpu.sync_copy(iid_hbm.at[wid], iid)

    seven = jnp.int32(7)
    lane = lax.iota(jnp.int32, 16)
    perms = [lane ^ s for s in (1, 2, 4, 8)]
    picks = [(lane & s) == 0 for s in (1, 2, 4, 8)]

    def _take(v, idx):
        return jnp.take_along_axis(v, idx, axis=0)

    def grp_body(g, carry):
        j = g // (IDXW // GRP)
        off = (g % (IDXW // GRP)) * GRP
        uvs = [uid[j, pl.ds(off + 16 * t, 16)] for t in range(GRP // 16)]
        ivs = [iid[j, pl.ds(off + 16 * t, 16)] for t in range(GRP // 16)]
        for t in range(GRP // 16):
            for r in range(16):
                slot = 16 * t + r
                u0 = pl.multiple_of((uvs[t][r] >> 3) << 3, 8)
                i0 = pl.multiple_of((ivs[t][r] >> 3) << 3, 8)
                pltpu.async_copy(p_hbm.at[pl.ds(u0, 8)], pg.at[slot], sem)
                pltpu.async_copy(q_hbm.at[pl.ds(i0, 8)], qg.at[slot], sem)
        for slot in range(GRP):
            pltpu.make_async_copy(p_hbm.at[pl.ds(0, 8)], pg.at[slot], sem).wait()
            pltpu.make_async_copy(q_hbm.at[pl.ds(0, 8)], qg.at[slot], sem).wait()
        for t in range(GRP // 16):
            vs = []
            for r in range(16):
                slot = 16 * t + r
                ur = uvs[t][r] & seven
                ir = ivs[t][r] & seven
                acc = pg[slot, ur, pl.ds(0, 16)] * qg[slot, ir, pl.ds(0, 16)]
                for k in range(1, LATENT // 16):
                    acc = acc + (pg[slot, ur, pl.ds(16 * k, 16)]
                                 * qg[slot, ir, pl.ds(16 * k, 16)])
                vs.append(acc)
            for lvl in range(4):
                nxt = []
                for i in range(0, len(vs), 2):
                    a2, b2 = vs[i], vs[i + 1]
                    ap = a2 + _take(a2, perms[lvl])
                    bp = b2 + _take(b2, perms[lvl])
                    nxt.append(jnp.where(picks[lvl], ap, bp))
                vs = nxt
            outb[pl.ds(g * GRP + 16 * t, 16)] = vs[0]
        return carry

    lax.fori_loop(0, NGRP, grp_body, 0)

    pltpu.sync_copy(outb, out_hbm.at[pl.ds(wid * CHUNK, CHUNK)])


_bias_scratch = [
    pltpu.VMEM((NIDX, IDXW), jnp.int32),
    pltpu.VMEM((NIDX, IDXW), jnp.int32),
    pltpu.VMEM((NIDX, IDXW), jnp.float32),
    pltpu.VMEM((NIDX, IDXW), jnp.float32),
    pltpu.VMEM((CHUNK,), jnp.float32),
    pltpu.SemaphoreType.DMA,
]


def _bias_body(uid_hbm, iid_hbm, part_hbm, bu_hbm, bi_hbm, out_hbm,
               uid, iid, bu, bi, outb, sem):
    wid = lax.axis_index("s") * NC + lax.axis_index("c")
    base = wid * CHUNK

    pltpu.sync_copy(uid_hbm.at[wid], uid)
    pltpu.sync_copy(iid_hbm.at[wid], iid)
    pltpu.sync_copy(part_hbm.at[pl.ds(base, CHUNK)], outb)

    copies = []
    for j in range(NIDX):
        copies.append(pltpu.async_copy(bu_hbm.at[uid.at[j]], bu.at[j], sem))
        copies.append(pltpu.async_copy(bi_hbm.at[iid.at[j]], bi.at[j], sem))
    for c in copies:
        c.wait()

    def blk_body(blk, carry):
        j = blk // (IDXW // 16)
        off = (blk % (IDXW // 16)) * 16
        o = blk * 16
        outb[pl.ds(o, 16)] = (outb[pl.ds(o, 16)]
                              + bu[j, pl.ds(off, 16)] + bi[j, pl.ds(off, 16)])
        return carry

    lax.fori_loop(0, CHUNK // 16, blk_body, 0)

    pltpu.sync_copy(outb, out_hbm.at[pl.ds(base, CHUNK)])


_dot_kernel = functools.partial(
    pl.kernel,
    out_type=jax.ShapeDtypeStruct((BATCH,), jnp.float32),
    mesh=_mesh,
    scratch_types=_dot_scratch,
    compiler_params=pltpu.CompilerParams(use_tc_tiling_on_sc=True),
)(_dot_body)

_bias_kernel = functools.partial(
    pl.kernel,
    out_type=jax.ShapeDtypeStruct((BATCH,), jnp.float32),
    mesh=_mesh,
    scratch_types=_bias_scratch,
    compiler_params=pltpu.CompilerParams(use_tc_tiling_on_sc=False),
)(_bias_body)


def kernel(users, items, P, Q, user_bias, item_bias):
    uid = users.reshape(NW, NIDX, IDXW)
    iid = items.reshape(NW, NIDX, IDXW)
    part = _dot_kernel(uid, iid, P, Q)
    bu_flat = user_bias.reshape(-1)
    bi_flat = item_bias.reshape(-1)
    return _bias_kernel(uid, iid, part, bu_flat, bi_flat)
